# baseline (speedup 1.0000x reference)
import functools

import jax
import jax.numpy as jnp
from jax import lax
from jax.experimental import pallas as pl
from jax.experimental.pallas import tpu as pltpu
from jax.experimental.pallas import tpu_sc as plsc

_N_NODES = 10000
_D = 64
_E = 320000
_G = 256
_NC = 2
_NS = 16
_NW = _NC * _NS
_EPT = _E // _NW
_CH = 125
_NCHUNK = _EPT // _CH
_NBUF = 10
_BLK = 10000
_NBLK = _N_NODES // _BLK


@functools.partial(
    pl.kernel,
    out_type=jax.ShapeDtypeStruct((_NC, _N_NODES, _D), jnp.bfloat16),
    mesh=plsc.VectorSubcoreMesh(core_axis_name="c", subcore_axis_name="s"),
    compiler_params=pltpu.CompilerParams(use_tc_tiling_on_sc=False),
    scratch_types=[
        pltpu.VMEM_SHARED((_N_NODES, _D), jnp.bfloat16),
        pltpu.VMEM((_NCHUNK, _CH), jnp.int32),
        pltpu.VMEM((_NCHUNK, _CH), jnp.int32),
        pltpu.VMEM((_NBUF, _CH, _D), jnp.bfloat16),
        pltpu.SemaphoreType.DMA((_NBUF,)),
        pltpu.SemaphoreType.DMA((_NBUF,)),
    ],
)
def _edge_agg(msg_hbm, src_hbm, dst_hbm, out_hbm, acc, idx_s, idx_d, rows,
              gsem, ssem):
    c = lax.axis_index("c")
    s = lax.axis_index("s")
    wid = s * _NC + c

    pltpu.sync_copy(src_hbm.at[wid], idx_s)
    pltpu.sync_copy(dst_hbm.at[wid], idx_d)

    zero = jnp.zeros((32,), jnp.bfloat16)

    def _zero_row(i, carry):
        for k in range(_D // 32):
            rows[0, i, pl.ds(k * 32, 32)] = zero
        return carry

    lax.fori_loop(0, _CH, _zero_row, 0)
    for t in range(5):
        pltpu.sync_copy(rows.at[0], acc.at[pl.ds(s * 625 + t * _CH, _CH)])
    plsc.subcore_barrier()

    def _group(g, carry):
        for b in range(_NBUF):
            @pl.when(g > 0)
            def _():
                pltpu.make_async_copy(
                    rows.at[b], acc.at[idx_d.at[g * _NBUF + b - _NBUF]],
                    ssem.at[b]).wait()
            pltpu.async_copy(msg_hbm.at[idx_s.at[g * _NBUF + b]], rows.at[b],
                             gsem.at[b])
        for b in range(_NBUF):
            pltpu.make_async_copy(msg_hbm.at[idx_s.at[g * _NBUF + b]],
                                  rows.at[b], gsem.at[b]).wait()
            pltpu.async_copy(rows.at[b], acc.at[idx_d.at[g * _NBUF + b]],
                             ssem.at[b], add=True)
        return carry

    lax.fori_loop(0, _NCHUNK // _NBUF, _group, 0)
    for b in range(_NBUF):
        pltpu.make_async_copy(
            rows.at[b], acc.at[idx_d.at[_NCHUNK - _NBUF + b]], ssem.at[b]
        ).wait()
    plsc.subcore_barrier()

    pltpu.sync_copy(acc.at[pl.ds(s * 624, 624)],
                    out_hbm.at[c, pl.ds(s * 624, 624)])

    @pl.when(s == _NS - 1)
    def _():
        pltpu.sync_copy(acc.at[pl.ds(_NS * 624, _N_NODES - _NS * 624)],
                        out_hbm.at[c, pl.ds(_NS * 624, _N_NODES - _NS * 624)])


def _dot(a, b):
    return jnp.dot(a, b, preferred_element_type=jnp.float32)


def _embed_msg_body(x_ref, we_ref, be_ref, wm_ref, bm_ref, st_ref, msg_ref):
    s = jnp.maximum(_dot(x_ref[...], we_ref[...]) + be_ref[...], 0.0)
    st_ref[...] = s
    msg_ref[...] = jnp.maximum(_dot(s, wm_ref[...]) + bm_ref[...],
                               0.0).astype(jnp.bfloat16)


def _embed_msg(x, w_emb, b_emb, w_msg, b_msg):
    k = x.shape[1]
    return pl.pallas_call(
        _embed_msg_body,
        grid=(_NBLK,),
        in_specs=[
            pl.BlockSpec((_BLK, k), lambda i: (i, 0)),
            pl.BlockSpec((k, _D), lambda i: (0, 0)),
            pl.BlockSpec((1, _D), lambda i: (0, 0)),
            pl.BlockSpec((_D, _D), lambda i: (0, 0)),
            pl.BlockSpec((1, _D), lambda i: (0, 0)),
        ],
        out_specs=[
            pl.BlockSpec((_BLK, _D), lambda i: (i, 0)),
            pl.BlockSpec((_BLK, _D), lambda i: (i, 0)),
        ],
        out_shape=[
            jax.ShapeDtypeStruct((_N_NODES, _D), jnp.float32),
            jax.ShapeDtypeStruct((_N_NODES, _D), jnp.bfloat16),
        ],
    )(x, w_emb, b_emb, w_msg, b_msg)


def _upd_msg_body(st_ref, p_ref, wu_ref, bu_ref, wm_ref, bm_ref,
                  sto_ref, msg_ref):
    agg = p_ref[0].astype(jnp.float32) + p_ref[1].astype(jnp.float32)
    s = st_ref[...] + jnp.maximum(_dot(agg, wu_ref[...]) + bu_ref[...], 0.0)
    sto_ref[...] = s
    msg_ref[...] = jnp.maximum(_dot(s, wm_ref[...]) + bm_ref[...],
                               0.0).astype(jnp.bfloat16)


def _upd_msg(state, partials, w_upd, b_upd, w_msg, b_msg):
    return pl.pallas_call(
        _upd_msg_body,
        grid=(_NBLK,),
        in_specs=[
            pl.BlockSpec((_BLK, _D), lambda i: (i, 0)),
            pl.BlockSpec((_NC, _BLK, _D), lambda i: (0, i, 0)),
            pl.BlockSpec((_D, _D), lambda i: (0, 0)),
            pl.BlockSpec((1, _D), lambda i: (0, 0)),
            pl.BlockSpec((_D, _D), lambda i: (0, 0)),
            pl.BlockSpec((1, _D), lambda i: (0, 0)),
        ],
        out_specs=[
            pl.BlockSpec((_BLK, _D), lambda i: (i, 0)),
            pl.BlockSpec((_BLK, _D), lambda i: (i, 0)),
        ],
        out_shape=[
            jax.ShapeDtypeStruct((_N_NODES, _D), jnp.float32),
            jax.ShapeDtypeStruct((_N_NODES, _D), jnp.bfloat16),
        ],
    )(state, partials, w_upd, b_upd, w_msg, b_msg)


def _upd_pool_body(st_ref, p_ref, wu_ref, bu_ref, bt_ref, wo_ref, bo_ref,
                   o_ref, acc_ref):
    i = pl.program_id(0)

    @pl.when(i == 0)
    def _():
        acc_ref[...] = jnp.zeros_like(acc_ref)

    agg = p_ref[0].astype(jnp.float32) + p_ref[1].astype(jnp.float32)
    s = st_ref[...] + jnp.maximum(_dot(agg, wu_ref[...]) + bu_ref[...], 0.0)
    b = bt_ref[0, 0, :]
    onehot = (b[:, None] == lax.broadcasted_iota(jnp.int32, (1, _G), 1)
              ).astype(jnp.float32)
    acc_ref[...] += lax.dot_general(
        onehot, s, (((0,), (0,)), ((), ())),
        preferred_element_type=jnp.float32)

    @pl.when(i == pl.num_programs(0) - 1)
    def _():
        o_ref[...] = _dot(acc_ref[...], wo_ref[...]) + bo_ref[...]


def _upd_pool(state, partials, w_upd, b_upd, batch3, w_out, b_out):
    m = w_out.shape[1]
    return pl.pallas_call(
        _upd_pool_body,
        grid=(_NBLK,),
        in_specs=[
            pl.BlockSpec((_BLK, _D), lambda i: (i, 0)),
            pl.BlockSpec((_NC, _BLK, _D), lambda i: (0, i, 0)),
            pl.BlockSpec((_D, _D), lambda i: (0, 0)),
            pl.BlockSpec((1, _D), lambda i: (0, 0)),
            pl.BlockSpec((1, 1, _BLK), lambda i: (i, 0, 0)),
            pl.BlockSpec((_D, m), lambda i: (0, 0)),
            pl.BlockSpec((1, m), lambda i: (0, 0)),
        ],
        out_specs=pl.BlockSpec((_G, m), lambda i: (0, 0)),
        out_shape=jax.ShapeDtypeStruct((_G, m), jnp.float32),
        scratch_shapes=[pltpu.VMEM((_G, _D), jnp.float32)],
    )(state, partials, w_upd, b_upd, batch3, w_out, b_out)


def kernel(x, edge_index, batch, W_emb, b_emb, W_msg, b_msg, W_upd, b_upd,
           W_out, b_out):
    src = edge_index[0].astype(jnp.int32).reshape(_NW, _NCHUNK, _CH)
    dst = edge_index[1].astype(jnp.int32).reshape(_NW, _NCHUNK, _CH)
    batch3 = batch.astype(jnp.int32).reshape(_NBLK, 1, _BLK)
    n_rounds = W_msg.shape[0]

    state, msg = _embed_msg(x, W_emb, b_emb.reshape(1, -1),
                            W_msg[0], b_msg[0].reshape(1, -1))
    for r in range(n_rounds - 1):
        partials = _edge_agg(msg, src, dst)
        state, msg = _upd_msg(state, partials, W_upd[r],
                              b_upd[r].reshape(1, -1),
                              W_msg[r + 1], b_msg[r + 1].reshape(1, -1))
    partials = _edge_agg(msg, src, dst)
    return _upd_pool(state, partials, W_upd[n_rounds - 1],
                     b_upd[n_rounds - 1].reshape(1, -1),
                     batch3, W_out, b_out.reshape(1, -1))


# speedup vs baseline: 1.0168x; 1.0168x over previous
import functools

import jax
import jax.numpy as jnp
from jax import lax
from jax.experimental import pallas as pl
from jax.experimental.pallas import tpu as pltpu
from jax.experimental.pallas import tpu_sc as plsc

_N_NODES = 10000
_D = 64
_E = 320000
_G = 256
_NC = 2
_NS = 16
_NW = _NC * _NS
_EPT = _E // _NW
_CH = 125
_NCHUNK = _EPT // _CH
_NBUF = 10
_BLK = 5000
_NBLK = _N_NODES // _BLK


@functools.partial(
    pl.kernel,
    out_type=jax.ShapeDtypeStruct((_NC, _N_NODES, _D), jnp.bfloat16),
    mesh=plsc.VectorSubcoreMesh(core_axis_name="c", subcore_axis_name="s"),
    compiler_params=pltpu.CompilerParams(use_tc_tiling_on_sc=False),
    scratch_types=[
        pltpu.VMEM_SHARED((_N_NODES, _D), jnp.bfloat16),
        pltpu.VMEM((_NCHUNK, _CH), jnp.int32),
        pltpu.VMEM((_NCHUNK, _CH), jnp.int32),
        pltpu.VMEM((_NBUF, _CH, _D), jnp.bfloat16),
        pltpu.SemaphoreType.DMA((_NBUF,)),
        pltpu.SemaphoreType.DMA((_NBUF,)),
    ],
)
def _edge_agg(msg_hbm, src_hbm, dst_hbm, out_hbm, acc, idx_s, idx_d, rows,
              gsem, ssem):
    c = lax.axis_index("c")
    s = lax.axis_index("s")
    wid = s * _NC + c

    pltpu.sync_copy(src_hbm.at[wid], idx_s)
    pltpu.sync_copy(dst_hbm.at[wid], idx_d)

    zero = jnp.zeros((32,), jnp.bfloat16)

    def _zero_row(i, carry):
        for k in range(_D // 32):
            rows[0, i, pl.ds(k * 32, 32)] = zero
        return carry

    lax.fori_loop(0, _CH, _zero_row, 0)
    for t in range(5):
        pltpu.sync_copy(rows.at[0], acc.at[pl.ds(s * 625 + t * _CH, _CH)])
    plsc.subcore_barrier()

    def _group(g, carry):
        for b in range(_NBUF):
            @pl.when(g > 0)
            def _():
                pltpu.make_async_copy(
                    rows.at[b], acc.at[idx_d.at[g * _NBUF + b - _NBUF]],
                    ssem.at[b]).wait()
            pltpu.async_copy(msg_hbm.at[idx_s.at[g * _NBUF + b]], rows.at[b],
                             gsem.at[b])
        for b in range(_NBUF):
            pltpu.make_async_copy(msg_hbm.at[idx_s.at[g * _NBUF + b]],
                                  rows.at[b], gsem.at[b]).wait()
            pltpu.async_copy(rows.at[b], acc.at[idx_d.at[g * _NBUF + b]],
                             ssem.at[b], add=True)
        return carry

    lax.fori_loop(0, _NCHUNK // _NBUF, _group, 0)
    for b in range(_NBUF):
        pltpu.make_async_copy(
            rows.at[b], acc.at[idx_d.at[_NCHUNK - _NBUF + b]], ssem.at[b]
        ).wait()
    plsc.subcore_barrier()

    pltpu.sync_copy(acc.at[pl.ds(s * 624, 624)],
                    out_hbm.at[c, pl.ds(s * 624, 624)])

    @pl.when(s == _NS - 1)
    def _():
        pltpu.sync_copy(acc.at[pl.ds(_NS * 624, _N_NODES - _NS * 624)],
                        out_hbm.at[c, pl.ds(_NS * 624, _N_NODES - _NS * 624)])


def _dot(a, b):
    return jnp.dot(a, b, preferred_element_type=jnp.float32)


def _embed_msg_body(x_ref, we_ref, be_ref, wm_ref, bm_ref, st_ref, msg_ref):
    s = jnp.maximum(_dot(x_ref[...], we_ref[...]) + be_ref[...], 0.0)
    st_ref[...] = s
    msg_ref[...] = jnp.maximum(_dot(s, wm_ref[...]) + bm_ref[...],
                               0.0).astype(jnp.bfloat16)


def _embed_msg(x, w_emb, b_emb, w_msg, b_msg):
    k = x.shape[1]
    return pl.pallas_call(
        _embed_msg_body,
        grid=(_NBLK,),
        in_specs=[
            pl.BlockSpec((_BLK, k), lambda i: (i, 0)),
            pl.BlockSpec((k, _D), lambda i: (0, 0)),
            pl.BlockSpec((1, _D), lambda i: (0, 0)),
            pl.BlockSpec((_D, _D), lambda i: (0, 0)),
            pl.BlockSpec((1, _D), lambda i: (0, 0)),
        ],
        out_specs=[
            pl.BlockSpec((_BLK, _D), lambda i: (i, 0)),
            pl.BlockSpec((_BLK, _D), lambda i: (i, 0)),
        ],
        out_shape=[
            jax.ShapeDtypeStruct((_N_NODES, _D), jnp.float32),
            jax.ShapeDtypeStruct((_N_NODES, _D), jnp.bfloat16),
        ],
    )(x, w_emb, b_emb, w_msg, b_msg)


def _upd_msg_body(st_ref, p_ref, wu_ref, bu_ref, wm_ref, bm_ref,
                  sto_ref, msg_ref):
    agg = p_ref[0].astype(jnp.float32) + p_ref[1].astype(jnp.float32)
    s = st_ref[...] + jnp.maximum(_dot(agg, wu_ref[...]) + bu_ref[...], 0.0)
    sto_ref[...] = s
    msg_ref[...] = jnp.maximum(_dot(s, wm_ref[...]) + bm_ref[...],
                               0.0).astype(jnp.bfloat16)


def _upd_msg(state, partials, w_upd, b_upd, w_msg, b_msg):
    return pl.pallas_call(
        _upd_msg_body,
        grid=(_NBLK,),
        in_specs=[
            pl.BlockSpec((_BLK, _D), lambda i: (i, 0)),
            pl.BlockSpec((_NC, _BLK, _D), lambda i: (0, i, 0)),
            pl.BlockSpec((_D, _D), lambda i: (0, 0)),
            pl.BlockSpec((1, _D), lambda i: (0, 0)),
            pl.BlockSpec((_D, _D), lambda i: (0, 0)),
            pl.BlockSpec((1, _D), lambda i: (0, 0)),
        ],
        out_specs=[
            pl.BlockSpec((_BLK, _D), lambda i: (i, 0)),
            pl.BlockSpec((_BLK, _D), lambda i: (i, 0)),
        ],
        out_shape=[
            jax.ShapeDtypeStruct((_N_NODES, _D), jnp.float32),
            jax.ShapeDtypeStruct((_N_NODES, _D), jnp.bfloat16),
        ],
    )(state, partials, w_upd, b_upd, w_msg, b_msg)


def _upd_pool_body(st_ref, p_ref, wu_ref, bu_ref, bt_ref, wo_ref, bo_ref,
                   o_ref, acc_ref):
    i = pl.program_id(0)

    @pl.when(i == 0)
    def _():
        acc_ref[...] = jnp.zeros_like(acc_ref)

    agg = p_ref[0].astype(jnp.float32) + p_ref[1].astype(jnp.float32)
    s = st_ref[...] + jnp.maximum(_dot(agg, wu_ref[...]) + bu_ref[...], 0.0)
    b = bt_ref[0, 0, :]
    onehot = (b[:, None] == lax.broadcasted_iota(jnp.int32, (1, _G), 1)
              ).astype(jnp.float32)
    acc_ref[...] += lax.dot_general(
        onehot, s, (((0,), (0,)), ((), ())),
        preferred_element_type=jnp.float32)

    @pl.when(i == pl.num_programs(0) - 1)
    def _():
        o_ref[...] = _dot(acc_ref[...], wo_ref[...]) + bo_ref[...]


def _upd_pool(state, partials, w_upd, b_upd, batch3, w_out, b_out):
    m = w_out.shape[1]
    return pl.pallas_call(
        _upd_pool_body,
        grid=(_NBLK,),
        in_specs=[
            pl.BlockSpec((_BLK, _D), lambda i: (i, 0)),
            pl.BlockSpec((_NC, _BLK, _D), lambda i: (0, i, 0)),
            pl.BlockSpec((_D, _D), lambda i: (0, 0)),
            pl.BlockSpec((1, _D), lambda i: (0, 0)),
            pl.BlockSpec((1, 1, _BLK), lambda i: (i, 0, 0)),
            pl.BlockSpec((_D, m), lambda i: (0, 0)),
            pl.BlockSpec((1, m), lambda i: (0, 0)),
        ],
        out_specs=pl.BlockSpec((_G, m), lambda i: (0, 0)),
        out_shape=jax.ShapeDtypeStruct((_G, m), jnp.float32),
        scratch_shapes=[pltpu.VMEM((_G, _D), jnp.float32)],
    )(state, partials, w_upd, b_upd, batch3, w_out, b_out)


def kernel(x, edge_index, batch, W_emb, b_emb, W_msg, b_msg, W_upd, b_upd,
           W_out, b_out):
    src = edge_index[0].astype(jnp.int32).reshape(_NW, _NCHUNK, _CH)
    dst = edge_index[1].astype(jnp.int32).reshape(_NW, _NCHUNK, _CH)
    batch3 = batch.astype(jnp.int32).reshape(_NBLK, 1, _BLK)
    n_rounds = W_msg.shape[0]

    state, msg = _embed_msg(x, W_emb, b_emb.reshape(1, -1),
                            W_msg[0], b_msg[0].reshape(1, -1))
    for r in range(n_rounds - 1):
        partials = _edge_agg(msg, src, dst)
        state, msg = _upd_msg(state, partials, W_upd[r],
                              b_upd[r].reshape(1, -1),
                              W_msg[r + 1], b_msg[r + 1].reshape(1, -1))
    partials = _edge_agg(msg, src, dst)
    return _upd_pool(state, partials, W_upd[n_rounds - 1],
                     b_upd[n_rounds - 1].reshape(1, -1),
                     batch3, W_out, b_out.reshape(1, -1))
